# baseline (speedup 1.0000x reference)
import functools

import jax
import jax.numpy as jnp
from jax import lax
from jax.experimental import pallas as pl
from jax.experimental.pallas import tpu as pltpu
from jax.experimental.pallas import tpu_sc as plsc

H = 1024
NV = H // 16
CH = 16
EPS = 1e-12
MAGIC = 0x5F3759DF


def _lane_sum(x):
    for k in (8, 4, 2, 1):
        idx = lax.iota(jnp.int32, 16) ^ k
        perm = lax.gather(
            x, idx[:, None],
            lax.GatherDimensionNumbers(
                offset_dims=(), collapsed_slice_dims=(0,),
                start_index_map=(0,)),
            slice_sizes=(1,),
            mode=lax.GatherScatterMode.PROMISE_IN_BOUNDS)
        x = x + perm
    return x


def _rsqrt_scalar(v):
    i = lax.bitcast_convert_type(v, jnp.int32)
    i = MAGIC - lax.shift_right_logical(i, 1)
    y = lax.bitcast_convert_type(i, jnp.float32)
    for _ in range(3):
        y = y * (1.5 - 0.5 * v * y * y)
    return y


def _make_sc_call(n_rows, rows_per_worker):
    n_chunks = rows_per_worker // CH
    chunks_per_batch = 512 // CH
    mesh = plsc.VectorSubcoreMesh(core_axis_name="c", subcore_axis_name="s")

    @functools.partial(
        pl.kernel,
        mesh=mesh,
        out_type=jax.ShapeDtypeStruct((n_rows, H), jnp.float32),
        scratch_types=[
            pltpu.VMEM((rows_per_worker,), jnp.int32),
            pltpu.VMEM((rows_per_worker,), jnp.int32),
            pltpu.VMEM((CH, H), jnp.float32),
            pltpu.VMEM((CH, H), jnp.float32),
            pltpu.VMEM((CH, H), jnp.float32),
            pltpu.VMEM((CH, H), jnp.float32),
            pltpu.VMEM((H,), jnp.float32),
            pltpu.VMEM((H,), jnp.float32),
            pltpu.SemaphoreType.DMA,
            pltpu.SemaphoreType.DMA,
        ],
    )
    def sc_call(pa_hbm, sp_hbm, pos_hbm, a_hbm, b_hbm, w_hbm, bias_hbm,
                out_hbm, pa_v, sp_v, a_rows, b_rows, pos_rows, obuf,
                w_v, bias_v, sem_a, sem_b):
        wid = lax.axis_index("s") * 2 + lax.axis_index("c")
        base = wid * rows_per_worker
        pltpu.sync_copy(pa_hbm.at[pl.ds(base, rows_per_worker)], pa_v)
        pltpu.sync_copy(sp_hbm.at[pl.ds(base, rows_per_worker)], sp_v)
        pltpu.sync_copy(w_hbm, w_v)
        pltpu.sync_copy(bias_hbm, bias_v)

        def chunk_body(c, carry):
            s0 = lax.rem(c, chunks_per_batch) * CH
            ipa = pa_v[pl.ds(c * CH, CH)]
            isp = sp_v[pl.ds(c * CH, CH)]
            da = pltpu.async_copy(a_hbm.at[ipa], a_rows, sem_a)
            db = pltpu.async_copy(b_hbm.at[isp], b_rows, sem_b)
            pltpu.sync_copy(pos_hbm.at[pl.ds(s0, CH)], pos_rows)
            da.wait()
            db.wait()

            def row_body(r, rcarry):
                acc = jnp.zeros((16,), jnp.float32)
                acc2 = jnp.zeros((16,), jnp.float32)
                for v in range(NV):
                    sl = pl.ds(v * 16, 16)
                    x = a_rows[r, sl] + b_rows[r, sl] + pos_rows[r, sl]
                    obuf[r, sl] = x
                    acc = acc + x
                    acc2 = acc2 + x * x
                mu = _lane_sum(acc) * (1.0 / H)
                q = _lane_sum(acc2) * (1.0 / H)
                var = q - mu * mu
                rstd = _rsqrt_scalar(var[0] + EPS)
                for v in range(NV):
                    sl = pl.ds(v * 16, 16)
                    xn = (obuf[r, sl] - mu) * rstd
                    obuf[r, sl] = xn * w_v[sl] + bias_v[sl]
                return rcarry

            lax.fori_loop(0, CH, row_body, 0)
            pltpu.sync_copy(obuf, out_hbm.at[pl.ds(base + c * CH, CH)])
            return carry

        lax.fori_loop(0, n_chunks, chunk_body, 0)

    return sc_call


def kernel(top_vecs, sent_struct_vec, pos_table, a_table, b_table, ln_w, ln_b):
    b, s, h = top_vecs.shape
    ssv = sent_struct_vec.astype(jnp.int32)
    pa = ssv[:, :, 0].reshape(-1)
    sp = ssv[:, :, 1].reshape(-1)
    n_rows = b * s
    sc_call = _make_sc_call(n_rows, n_rows // 32)
    out = sc_call(pa, sp, pos_table, a_table, b_table, ln_w, ln_b)
    return out.reshape(b, s, h)


# speedup vs baseline: 1.5111x; 1.5111x over previous
import functools

import jax
import jax.numpy as jnp
from jax import lax
from jax.experimental import pallas as pl
from jax.experimental.pallas import tpu as pltpu
from jax.experimental.pallas import tpu_sc as plsc

H = 1024
NV = H // 16
CH = 16
EPS = 1e-12
MAGIC = 0x5F3759DF


def _lane_sum(x):
    for k in (8, 4, 2, 1):
        idx = lax.iota(jnp.int32, 16) ^ k
        perm = lax.gather(
            x, idx[:, None],
            lax.GatherDimensionNumbers(
                offset_dims=(), collapsed_slice_dims=(0,),
                start_index_map=(0,)),
            slice_sizes=(1,),
            mode=lax.GatherScatterMode.PROMISE_IN_BOUNDS)
        x = x + perm
    return x


def _rsqrt_scalar(v):
    i = lax.bitcast_convert_type(v, jnp.int32)
    i = MAGIC - lax.shift_right_logical(i, 1)
    y = lax.bitcast_convert_type(i, jnp.float32)
    for _ in range(3):
        y = y * (1.5 - 0.5 * v * y * y)
    return y


def _make_sc_call(n_rows, rows_per_worker):
    n_chunks = rows_per_worker // CH
    n_groups = n_chunks // 2
    chunks_per_batch = 512 // CH
    mesh = plsc.VectorSubcoreMesh(core_axis_name="c", subcore_axis_name="s")

    @functools.partial(
        pl.kernel,
        mesh=mesh,
        out_type=jax.ShapeDtypeStruct((n_rows, H), jnp.float32),
        scratch_types=[
            pltpu.VMEM((rows_per_worker,), jnp.int32),
            pltpu.VMEM((rows_per_worker,), jnp.int32),
            pltpu.VMEM((CH, H), jnp.float32),
            pltpu.VMEM((CH, H), jnp.float32),
            pltpu.VMEM((CH, H), jnp.float32),
            pltpu.VMEM((CH, H), jnp.float32),
            pltpu.VMEM((CH, H), jnp.float32),
            pltpu.VMEM((CH, H), jnp.float32),
            pltpu.VMEM((CH, H), jnp.float32),
            pltpu.VMEM((H,), jnp.float32),
            pltpu.VMEM((H,), jnp.float32),
            pltpu.SMEM((CH,), jnp.float32),
            pltpu.SMEM((CH,), jnp.float32),
            pltpu.SemaphoreType.DMA,
            pltpu.SemaphoreType.DMA,
            pltpu.SemaphoreType.DMA,
            pltpu.SemaphoreType.DMA,
            pltpu.SemaphoreType.DMA,
            pltpu.SemaphoreType.DMA,
        ],
    )
    def sc_call(pa_hbm, sp_hbm, pos_hbm, a_hbm, b_hbm, w_hbm, bias_hbm,
                out_hbm, pa_v, sp_v, a0, a1, b0, b1, posb, o0, o1,
                w_v, bias_v, mu_sm, rs_sm,
                sa0, sa1, sb0, sb1, so0, so1):
        wid = lax.axis_index("s") * 2 + lax.axis_index("c")
        base = wid * rows_per_worker
        pltpu.sync_copy(pa_hbm.at[pl.ds(base, rows_per_worker)], pa_v)
        pltpu.sync_copy(sp_hbm.at[pl.ds(base, rows_per_worker)], sp_v)
        pltpu.sync_copy(w_hbm, w_v)
        pltpu.sync_copy(bias_hbm, bias_v)

        def start_gather(c, ab, bb, sa, sb):
            ipa = pa_v[pl.ds(c * CH, CH)]
            isp = sp_v[pl.ds(c * CH, CH)]
            pltpu.async_copy(a_hbm.at[ipa], ab, sa)
            pltpu.async_copy(b_hbm.at[isp], bb, sb)

        def wait_gather(ab, bb, sa, sb):
            ipa = pa_v[pl.ds(0, CH)]
            pltpu.make_async_copy(a_hbm.at[ipa], ab, sa).wait()
            pltpu.make_async_copy(b_hbm.at[ipa], bb, sb).wait()

        def compute_chunk(c, ar, br, ob):
            s0 = lax.rem(c, chunks_per_batch) * CH
            pltpu.sync_copy(pos_hbm.at[pl.ds(s0, CH)], posb)

            def row_body(r, rcarry):
                acc = jnp.zeros((16,), jnp.float32)
                acc2 = jnp.zeros((16,), jnp.float32)
                for v in range(NV):
                    sl = pl.ds(v * 16, 16)
                    x = ar[r, sl] + br[r, sl] + posb[r, sl]
                    ob[r, sl] = x
                    acc = acc + x
                    acc2 = acc2 + x * x
                mu = _lane_sum(acc) * (1.0 / H)
                q = _lane_sum(acc2) * (1.0 / H)
                var = q - mu * mu
                mu_sm[r] = mu[0]
                rs_sm[r] = _rsqrt_scalar(var[0] + EPS)
                return rcarry

            lax.fori_loop(0, CH, row_body, 0)

            def col_body(v, vcarry):
                sl = pl.ds(v * 16, 16)
                wv = w_v[sl]
                bv = bias_v[sl]
                for r in range(CH):
                    x = ob[r, sl]
                    ob[r, sl] = (x - mu_sm[r]) * rs_sm[r] * wv + bv
                return vcarry

            lax.fori_loop(0, NV, col_body, 0)

        def wait_out(ob, so):
            pltpu.make_async_copy(ob, out_hbm.at[pl.ds(base, CH)], so).wait()

        start_gather(0, a0, b0, sa0, sb0)

        def group_body(g, carry):
            c0 = 2 * g
            c1 = c0 + 1
            start_gather(c1, a1, b1, sa1, sb1)
            wait_gather(a0, b0, sa0, sb0)

            @pl.when(g > 0)
            def _():
                wait_out(o0, so0)

            compute_chunk(c0, a0, b0, o0)
            pltpu.async_copy(o0, out_hbm.at[pl.ds(base + c0 * CH, CH)], so0)

            @pl.when(g < n_groups - 1)
            def _():
                start_gather(c0 + 2, a0, b0, sa0, sb0)

            wait_gather(a1, b1, sa1, sb1)

            @pl.when(g > 0)
            def _():
                wait_out(o1, so1)

            compute_chunk(c1, a1, b1, o1)
            pltpu.async_copy(o1, out_hbm.at[pl.ds(base + c1 * CH, CH)], so1)
            return carry

        lax.fori_loop(0, n_groups, group_body, 0)
        wait_out(o0, so0)
        wait_out(o1, so1)

    return sc_call


def kernel(top_vecs, sent_struct_vec, pos_table, a_table, b_table, ln_w, ln_b):
    b, s, h = top_vecs.shape
    ssv = sent_struct_vec.astype(jnp.int32)
    pa = ssv[:, :, 0].reshape(-1)
    sp = ssv[:, :, 1].reshape(-1)
    n_rows = b * s
    sc_call = _make_sc_call(n_rows, n_rows // 32)
    out = sc_call(pa, sp, pos_table, a_table, b_table, ln_w, ln_b)
    return out.reshape(b, s, h)


# speedup vs baseline: 1.5373x; 1.0173x over previous
import functools

import jax
import jax.numpy as jnp
from jax import lax
from jax.experimental import pallas as pl
from jax.experimental.pallas import tpu as pltpu
from jax.experimental.pallas import tpu_sc as plsc

H = 1024
NV = H // 16
CH = 16
EPS = 1e-12
MAGIC = 0x5F3759DF


def _lane_sum(x):
    for k in (8, 4, 2, 1):
        idx = lax.iota(jnp.int32, 16) ^ k
        perm = lax.gather(
            x, idx[:, None],
            lax.GatherDimensionNumbers(
                offset_dims=(), collapsed_slice_dims=(0,),
                start_index_map=(0,)),
            slice_sizes=(1,),
            mode=lax.GatherScatterMode.PROMISE_IN_BOUNDS)
        x = x + perm
    return x


def _rsqrt_scalar(v):
    i = lax.bitcast_convert_type(v, jnp.int32)
    i = MAGIC - lax.shift_right_logical(i, 1)
    y = lax.bitcast_convert_type(i, jnp.float32)
    for _ in range(3):
        y = y * (1.5 - 0.5 * v * y * y)
    return y


def _make_sc_call(n_rows, rows_per_worker):
    n_chunks = rows_per_worker // CH
    chunks_per_batch = 512 // CH
    n_groups = n_chunks // 2
    assert n_groups == chunks_per_batch
    mesh = plsc.VectorSubcoreMesh(core_axis_name="c", subcore_axis_name="s")

    @functools.partial(
        pl.kernel,
        mesh=mesh,
        out_type=jax.ShapeDtypeStruct((n_rows, H), jnp.float32),
        scratch_types=[
            pltpu.VMEM((rows_per_worker,), jnp.int32),
            pltpu.VMEM((rows_per_worker,), jnp.int32),
            pltpu.VMEM((CH, H), jnp.float32),
            pltpu.VMEM((CH, H), jnp.float32),
            pltpu.VMEM((CH, H), jnp.float32),
            pltpu.VMEM((CH, H), jnp.float32),
            pltpu.VMEM((CH, H), jnp.float32),
            pltpu.VMEM((CH, H), jnp.float32),
            pltpu.VMEM((CH, H), jnp.float32),
            pltpu.VMEM((H,), jnp.float32),
            pltpu.VMEM((H,), jnp.float32),
            pltpu.SMEM((CH,), jnp.float32),
            pltpu.SMEM((CH,), jnp.float32),
            pltpu.SemaphoreType.DMA,
            pltpu.SemaphoreType.DMA,
            pltpu.SemaphoreType.DMA,
            pltpu.SemaphoreType.DMA,
            pltpu.SemaphoreType.DMA,
            pltpu.SemaphoreType.DMA,
        ],
    )
    def sc_call(pa_hbm, sp_hbm, pos_hbm, a_hbm, b_hbm, w_hbm, bias_hbm,
                out_hbm, pa_v, sp_v, a0, a1, b0, b1, posb, o0, o1,
                w_v, bias_v, mu_sm, rs_sm,
                sa0, sa1, sb0, sb1, so0, so1):
        wid = lax.axis_index("s") * 2 + lax.axis_index("c")
        base = wid * rows_per_worker
        pltpu.sync_copy(pa_hbm.at[pl.ds(base, rows_per_worker)], pa_v)
        pltpu.sync_copy(sp_hbm.at[pl.ds(base, rows_per_worker)], sp_v)
        pltpu.sync_copy(w_hbm, w_v)
        pltpu.sync_copy(bias_hbm, bias_v)

        def start_gather(c, ab, bb, sa, sb):
            ipa = pa_v[pl.ds(c * CH, CH)]
            isp = sp_v[pl.ds(c * CH, CH)]
            pltpu.async_copy(a_hbm.at[ipa], ab, sa)
            pltpu.async_copy(b_hbm.at[isp], bb, sb)

        def wait_gather(ab, bb, sa, sb):
            ipa = pa_v[pl.ds(0, CH)]
            pltpu.make_async_copy(a_hbm.at[ipa], ab, sa).wait()
            pltpu.make_async_copy(b_hbm.at[ipa], bb, sb).wait()

        def compute_chunk(c, ar, br, ob):

            def row_body(r, rcarry):
                a4 = [jnp.zeros((16,), jnp.float32) for _ in range(4)]
                q4 = [jnp.zeros((16,), jnp.float32) for _ in range(4)]
                for v in range(NV):
                    sl = pl.ds(v * 16, 16)
                    x = ar[r, sl] + br[r, sl] + posb[r, sl]
                    ob[r, sl] = x
                    a4[v % 4] = a4[v % 4] + x
                    q4[v % 4] = q4[v % 4] + x * x
                acc = (a4[0] + a4[1]) + (a4[2] + a4[3])
                acc2 = (q4[0] + q4[1]) + (q4[2] + q4[3])
                mu = _lane_sum(acc) * (1.0 / H)
                q = _lane_sum(acc2) * (1.0 / H)
                var = q - mu * mu
                mu_sm[r] = mu[0]
                rs_sm[r] = _rsqrt_scalar(var[0] + EPS)
                return rcarry

            lax.fori_loop(0, CH, row_body, 0)

            def col_body(v, vcarry):
                sl = pl.ds(v * 16, 16)
                wv = w_v[sl]
                bv = bias_v[sl]
                for r in range(CH):
                    x = ob[r, sl]
                    ob[r, sl] = (x - mu_sm[r]) * rs_sm[r] * wv + bv
                return vcarry

            lax.fori_loop(0, NV, col_body, 0)

        def wait_out(ob, so):
            pltpu.make_async_copy(ob, out_hbm.at[pl.ds(base, CH)], so).wait()

        start_gather(0, a0, b0, sa0, sb0)

        def group_body(g, carry):
            c0 = g
            c1 = g + chunks_per_batch
            start_gather(c1, a1, b1, sa1, sb1)
            pltpu.sync_copy(pos_hbm.at[pl.ds(g * CH, CH)], posb)
            wait_gather(a0, b0, sa0, sb0)

            @pl.when(g > 0)
            def _():
                wait_out(o0, so0)

            compute_chunk(c0, a0, b0, o0)
            pltpu.async_copy(o0, out_hbm.at[pl.ds(base + c0 * CH, CH)], so0)

            @pl.when(g < n_groups - 1)
            def _():
                start_gather(g + 1, a0, b0, sa0, sb0)

            wait_gather(a1, b1, sa1, sb1)

            @pl.when(g > 0)
            def _():
                wait_out(o1, so1)

            compute_chunk(c1, a1, b1, o1)
            pltpu.async_copy(o1, out_hbm.at[pl.ds(base + c1 * CH, CH)], so1)
            return carry

        lax.fori_loop(0, n_groups, group_body, 0)
        wait_out(o0, so0)
        wait_out(o1, so1)

    return sc_call


def kernel(top_vecs, sent_struct_vec, pos_table, a_table, b_table, ln_w, ln_b):
    b, s, h = top_vecs.shape
    ssv = sent_struct_vec.astype(jnp.int32)
    pa = ssv[:, :, 0].reshape(-1)
    sp = ssv[:, :, 1].reshape(-1)
    n_rows = b * s
    sc_call = _make_sc_call(n_rows, n_rows // 32)
    out = sc_call(pa, sp, pos_table, a_table, b_table, ln_w, ln_b)
    return out.reshape(b, s, h)


# speedup vs baseline: 1.8600x; 1.2099x over previous
import functools

import jax
import jax.numpy as jnp
from jax import lax
from jax.experimental import pallas as pl
from jax.experimental.pallas import tpu as pltpu
from jax.experimental.pallas import tpu_sc as plsc

H = 1024
NV = H // 16
CH = 16
EPS = 1e-12


def _tc_stats_kernel(pos_ref, a_ref, b_ref, pa_ref, sp_ref, mu_ref, rs_ref):
    pos = pos_ref[...]
    at = a_ref[...]
    bt = b_ref[...]
    n_batch = pa_ref.shape[0]
    na = at.shape[0]

    s_pos = jnp.sum(pos, axis=1)
    q_pos = jnp.sum(pos * pos, axis=1)
    s_a = jnp.sum(at, axis=1)
    q_a = jnp.sum(at * at, axis=1)
    s_b = jnp.sum(bt, axis=1)
    q_b = jnp.sum(bt * bt, axis=1)
    dims = (((1,), (1,)), ((), ()))
    c_pa = lax.dot_general(pos, at, dims, preferred_element_type=jnp.float32)
    c_pb = lax.dot_general(pos, bt, dims, preferred_element_type=jnp.float32)
    c_ab = lax.dot_general(at, bt, dims, preferred_element_type=jnp.float32)
    iota_a = lax.broadcasted_iota(jnp.int32, (1, na), 1)

    def batch_body(b, carry):
        pa_b = pa_ref[b, :]
        sp_b = sp_ref[b, :]
        oha = (pa_b[:, None] == iota_a).astype(jnp.float32)
        ohb = (sp_b[:, None] == iota_a).astype(jnp.float32)
        s_ab = oha @ s_a + ohb @ s_b
        q_ab = oha @ q_a + ohb @ q_b
        t_pa = jnp.sum(oha * c_pa, axis=1)
        t_pb = jnp.sum(ohb * c_pb, axis=1)
        d = lax.dot_general(ohb, c_ab, dims,
                            preferred_element_type=jnp.float32)
        t_ab = jnp.sum(oha * d, axis=1)
        mu = (s_pos + s_ab) * (1.0 / H)
        ex2 = (q_pos + q_ab + 2.0 * (t_pa + t_pb + t_ab)) * (1.0 / H)
        var = ex2 - mu * mu
        mu_ref[b, :] = mu
        rs_ref[b, :] = lax.rsqrt(var + EPS)
        return carry

    lax.fori_loop(0, n_batch, batch_body, 0)


def _tc_stats(pos_table, a_table, b_table, pa, sp):
    n_batch, n_sent = pa.shape
    return pl.pallas_call(
        _tc_stats_kernel,
        out_shape=(
            jax.ShapeDtypeStruct((n_batch, n_sent), jnp.float32),
            jax.ShapeDtypeStruct((n_batch, n_sent), jnp.float32),
        ),
    )(pos_table, a_table, b_table, pa, sp)


def _make_sc_call(n_rows, rows_per_worker):
    n_chunks = rows_per_worker // CH
    chunks_per_batch = 512 // CH
    n_groups = n_chunks // 2
    assert n_groups == chunks_per_batch
    mesh = plsc.VectorSubcoreMesh(core_axis_name="c", subcore_axis_name="s")

    @functools.partial(
        pl.kernel,
        mesh=mesh,
        out_type=jax.ShapeDtypeStruct((n_rows, H), jnp.float32),
        scratch_types=[
            pltpu.VMEM((rows_per_worker,), jnp.int32),
            pltpu.VMEM((rows_per_worker,), jnp.int32),
            pltpu.VMEM((rows_per_worker,), jnp.float32),
            pltpu.VMEM((rows_per_worker,), jnp.float32),
            pltpu.VMEM((CH, H), jnp.float32),
            pltpu.VMEM((CH, H), jnp.float32),
            pltpu.VMEM((CH, H), jnp.float32),
            pltpu.VMEM((CH, H), jnp.float32),
            pltpu.VMEM((CH, H), jnp.float32),
            pltpu.VMEM((CH, H), jnp.float32),
            pltpu.VMEM((CH, H), jnp.float32),
            pltpu.VMEM((H,), jnp.float32),
            pltpu.VMEM((H,), jnp.float32),
            pltpu.SemaphoreType.DMA,
            pltpu.SemaphoreType.DMA,
            pltpu.SemaphoreType.DMA,
            pltpu.SemaphoreType.DMA,
            pltpu.SemaphoreType.DMA,
            pltpu.SemaphoreType.DMA,
        ],
    )
    def sc_call(pa_hbm, sp_hbm, pos_hbm, a_hbm, b_hbm, w_hbm, bias_hbm,
                mu_hbm, rs_hbm, out_hbm,
                pa_v, sp_v, mu_v, rs_v, a0, a1, b0, b1, posb, o0, o1,
                w_v, bias_v, sa0, sa1, sb0, sb1, so0, so1):
        wid = lax.axis_index("s") * 2 + lax.axis_index("c")
        base = wid * rows_per_worker
        pltpu.sync_copy(pa_hbm.at[pl.ds(base, rows_per_worker)], pa_v)
        pltpu.sync_copy(sp_hbm.at[pl.ds(base, rows_per_worker)], sp_v)
        pltpu.sync_copy(mu_hbm.at[pl.ds(base, rows_per_worker)], mu_v)
        pltpu.sync_copy(rs_hbm.at[pl.ds(base, rows_per_worker)], rs_v)
        pltpu.sync_copy(w_hbm, w_v)
        pltpu.sync_copy(bias_hbm, bias_v)

        def start_gather(c, ab, bb, sa, sb):
            ipa = pa_v[pl.ds(c * CH, CH)]
            isp = sp_v[pl.ds(c * CH, CH)]
            pltpu.async_copy(a_hbm.at[ipa], ab, sa)
            pltpu.async_copy(b_hbm.at[isp], bb, sb)

        def wait_gather(ab, bb, sa, sb):
            ipa = pa_v[pl.ds(0, CH)]
            pltpu.make_async_copy(a_hbm.at[ipa], ab, sa).wait()
            pltpu.make_async_copy(b_hbm.at[ipa], bb, sb).wait()

        def compute_chunk(c, ar, br, ob):
            muv = mu_v[pl.ds(c * CH, CH)]
            rsv = rs_v[pl.ds(c * CH, CH)]
            mus = [muv[r] for r in range(CH)]
            rss = [rsv[r] for r in range(CH)]

            def col_body(v, vcarry):
                sl = pl.ds(v * 16, 16)
                wv = w_v[sl]
                bv = bias_v[sl]
                for r in range(CH):
                    x = ar[r, sl] + br[r, sl] + posb[r, sl]
                    ob[r, sl] = (x - mus[r]) * rss[r] * wv + bv
                return vcarry

            lax.fori_loop(0, NV, col_body, 0)

        def wait_out(ob, so):
            pltpu.make_async_copy(ob, out_hbm.at[pl.ds(base, CH)], so).wait()

        start_gather(0, a0, b0, sa0, sb0)

        def group_body(g, carry):
            c0 = g
            c1 = g + chunks_per_batch
            start_gather(c1, a1, b1, sa1, sb1)
            pltpu.sync_copy(pos_hbm.at[pl.ds(g * CH, CH)], posb)
            wait_gather(a0, b0, sa0, sb0)

            @pl.when(g > 0)
            def _():
                wait_out(o0, so0)

            compute_chunk(c0, a0, b0, o0)
            pltpu.async_copy(o0, out_hbm.at[pl.ds(base + c0 * CH, CH)], so0)

            @pl.when(g < n_groups - 1)
            def _():
                start_gather(g + 1, a0, b0, sa0, sb0)

            wait_gather(a1, b1, sa1, sb1)

            @pl.when(g > 0)
            def _():
                wait_out(o1, so1)

            compute_chunk(c1, a1, b1, o1)
            pltpu.async_copy(o1, out_hbm.at[pl.ds(base + c1 * CH, CH)], so1)
            return carry

        lax.fori_loop(0, n_groups, group_body, 0)
        wait_out(o0, so0)
        wait_out(o1, so1)

    return sc_call


def kernel(top_vecs, sent_struct_vec, pos_table, a_table, b_table, ln_w, ln_b):
    b, s, h = top_vecs.shape
    ssv = sent_struct_vec.astype(jnp.int32)
    pa = ssv[:, :, 0]
    sp = ssv[:, :, 1]
    mu, rstd = _tc_stats(pos_table, a_table, b_table, pa, sp)
    n_rows = b * s
    sc_call = _make_sc_call(n_rows, n_rows // 32)
    out = sc_call(pa.reshape(-1), sp.reshape(-1), pos_table, a_table,
                  b_table, ln_w, ln_b, mu.reshape(-1), rstd.reshape(-1))
    return out.reshape(b, s, h)
